# baseline (device time: 75342 ns/iter reference)
import functools

import jax
import jax.numpy as jnp
from jax import lax
from jax.experimental import pallas as pl
from jax.experimental.pallas import tpu as pltpu

N_DEV = 16
PLANE = 4
NZ = 4


def kernel(A, B):
    m, k = A.shape
    _, n = B.shape
    hn = n // 2
    hg = hn // 2
    qr_rows = m // PLANE
    h2 = qr_rows // 2
    h4 = qr_rows // 4

    def body(a_ref, b_ref, out_ref, acc_r, acc_l,
             pb00, pb01, pb10, pb11, sa_recv, sb_recv,
             p1_send_r, p1_recv_r, p1_send_l, p1_recv_l,
             p2_send, p2_recv,
             p3_send_r, p3_recv_r, p3_send_l, p3_recv_l):
        my = lax.axis_index("i")
        z = lax.div(my, PLANE)
        q = lax.rem(my, PLANE)
        qr = z * PLANE + lax.rem(q + 1, PLANE)
        ql = z * PLANE + lax.rem(q + PLANE - 1, PLANE)
        pz1 = q + PLANE * jnp.bitwise_xor(z, 1)
        pz2 = q + PLANE * jnp.bitwise_xor(z, 2)
        neighbors = (qr, ql, pz1, pz2)

        barrier = pltpu.get_barrier_semaphore()
        for nbr in neighbors:
            pl.semaphore_signal(barrier, inc=1, device_id=(nbr,),
                                device_id_type=pl.DeviceIdType.MESH)
        pl.semaphore_wait(barrier, len(neighbors))

        def rdma(src, dst, ssem, rsem, tgt):
            cp = pltpu.make_async_remote_copy(
                src_ref=src, dst_ref=dst, send_sem=ssem, recv_sem=rsem,
                device_id=(tgt,), device_id_type=pl.DeviceIdType.MESH,
            )
            cp.start()
            return cp

        b1 = jnp.bitwise_and(z, 1)
        b2 = jnp.bitwise_and(lax.div(z, 2), 1)
        keep1 = b1 * h2
        send1 = (1 - b1) * h2
        d_keep = b2 * h4
        d_send = (1 - b2) * h4
        keep2 = keep1 + d_keep
        send2 = keep1 + d_send

        accs = (acc_r, acc_l)
        pbufs = {(0, 0): pb00, (0, 1): pb01, (1, 0): pb10, (1, 1): pb11}
        p1s = (p1_send_r, p1_send_l)
        p1r = (p1_recv_r, p1_recv_l)
        p3s = (p3_send_r, p3_send_l)
        p3r = (p3_recv_r, p3_recv_l)
        tgt_ = (qr, ql)
        gcol = (pl.ds(0, hg), pl.ds(hg, hg))
        ocol = tuple(pl.ds(i * hn + g * hg, hg)
                     for i in range(2) for g in range(2))

        def chunk_q(i, s, delta):
            d = -s + delta if i == 0 else s + delta
            return lax.rem(q + d + 2 * PLANE, PLANE)

        base_r = lax.rem(q + 1, PLANE) * qr_rows
        base_l = lax.rem(q + PLANE - 1, PLANE) * qr_rows
        bases = (base_r, base_l)

        b_halves = [None, None]

        def compute_quarter(i, idx):
            a_q = a_ref[pl.ds(idx * qr_rows, qr_rows), :].astype(jnp.bfloat16)
            accs[i][pl.ds(idx * qr_rows, qr_rows), :] = jnp.dot(
                a_q, b_halves[i], preferred_element_type=jnp.float32
            ).astype(jnp.bfloat16)

        def p1_hop(i, g, s):
            src_q = chunk_q(i, s, 0)
            return rdma(
                accs[i].at[pl.ds(src_q * qr_rows, qr_rows), gcol[g]],
                pbufs[(i, g)].at[s],
                p1s[i].at[3 * g + s], p1r[i].at[3 * g + s], tgt_[i])

        def p1_add(i, g, s):
            dst_q = chunk_q(i, s, -1 if i == 0 else 1)
            rows = pl.ds(dst_q * qr_rows, qr_rows)
            accs[i][rows, gcol[g]] = (
                accs[i][rows, gcol[g]] + pbufs[(i, g)][s])

        def z_step(i, g, step, off, nrows, buf, tgt):
            src = accs[i].at[pl.ds(bases[i] + off, nrows), gcol[g]]
            dst = src if buf is None else buf
            return rdma(src, dst,
                        p2_send.at[4 * step + 2 * g + i],
                        p2_recv.at[4 * step + 2 * g + i], tgt)

        ag_offs = (keep2, send2, send1)
        ag_len = (h4, h4, h2)

        def ag_hop(i, g, c, s):
            nrows = ag_len[c]
            if s == 0:
                rows = pl.ds(bases[i] + ag_offs[c], nrows)
                src = accs[i].at[rows, gcol[g]]
            else:
                back = chunk_q(i, s - 1, 0)
                rows = pl.ds(back * qr_rows + ag_offs[c], nrows)
                src = out_ref.at[rows, ocol[2 * i + g]]
            return rdma(src, out_ref.at[rows, ocol[2 * i + g]],
                        p3s[i].at[9 * g + 3 * c + s],
                        p3r[i].at[9 * g + 3 * c + s], tgt_[i])

        def out_own(i, g, off, nrows):
            rows = pl.ds(bases[i] + off, nrows)
            out_ref[rows, ocol[2 * i + g]] = accs[i][rows, gcol[g]]

        hops = {}
        for i, cols in enumerate((slice(0, hn), slice(hn, n))):
            b_halves[i] = b_ref[:, cols].astype(jnp.bfloat16)
            compute_quarter(i, q)
            hops[(i, 0)] = p1_hop(i, 0, 0)

        za = {}
        for s in range(PLANE - 1):
            compute_quarter(0, chunk_q(0, s, -1))
            compute_quarter(1, chunk_q(1, s, 1))
            for i in range(2):
                if s == 0:
                    hops[(i, 1)] = p1_hop(i, 1, 0)
                hops[(i, 0)].wait()
                p1_add(i, 0, s)
                if s < PLANE - 2:
                    hops[(i, 0)] = p1_hop(i, 0, s + 1)
                else:
                    za[(i, 0)] = z_step(i, 0, 0, send1, h2,
                                        sa_recv.at[2 * 0 + i], pz1)
                if s > 0:
                    hops[(i, 1)].wait()
                    p1_add(i, 1, s - 1)
                    hops[(i, 1)] = p1_hop(i, 1, s)
        for i in range(2):
            hops[(i, 1)].wait()
            p1_add(i, 1, PLANE - 2)
            za[(i, 1)] = z_step(i, 1, 0, send1, h2, sa_recv.at[2 + i], pz1)

        zb, zc, zd = {}, {}, {}
        ag = {}
        for g in range(2):
            for i in range(2):
                za[(i, g)].wait()
                rows = pl.ds(bases[i] + keep1, h2)
                accs[i][rows, gcol[g]] = (
                    accs[i][rows, gcol[g]] + sa_recv[2 * g + i])
                zb[(i, g)] = z_step(i, g, 1, send2, h4,
                                    sb_recv.at[2 * g + i], pz2)
        for g in range(2):
            for i in range(2):
                zb[(i, g)].wait()
                rows = pl.ds(bases[i] + keep2, h4)
                accs[i][rows, gcol[g]] = jnp.maximum(
                    accs[i][rows, gcol[g]] + sb_recv[2 * g + i], 0.0)
                zc[(i, g)] = z_step(i, g, 2, keep2, h4, None, pz2)
                zd[(i, g)] = z_step(i, g, 3, keep2, h4, None, pz1)
                ag[(i, g, 0)] = ag_hop(i, g, 0, 0)
                out_own(i, g, keep2, h4)
            if g == 0:
                for i in range(2):
                    zc[(i, 0)].wait()
                    zd2 = z_step(i, 0, 4, send2, h4, None, pz1)
                    zd[(i, 0)] = (zd[(i, 0)], zd2)
                    ag[(i, 0, 1)] = ag_hop(i, 0, 1, 0)
                    out_own(i, 0, send2, h4)

        for i in range(2):
            d1, d2 = zd[(i, 0)]
            d1.wait()
            d2.wait()
            ag[(i, 0, 2)] = ag_hop(i, 0, 2, 0)
            out_own(i, 0, send1, h2)
        for i in range(2):
            ag[(i, 0, 0)].wait()
            ag[(i, 0, 0)] = ag_hop(i, 0, 0, 1)
        for i in range(2):
            zc[(i, 1)].wait()
            zd2 = z_step(i, 1, 4, send2, h4, None, pz1)
            zd[(i, 1)] = (zd[(i, 1)], zd2)
            ag[(i, 1, 1)] = ag_hop(i, 1, 1, 0)
            out_own(i, 1, send2, h4)
        for i in range(2):
            d1, d2 = zd[(i, 1)]
            d1.wait()
            d2.wait()
            ag[(i, 1, 2)] = ag_hop(i, 1, 2, 0)
            out_own(i, 1, send1, h2)

        hop_at = {(0, 0): 1, (0, 1): 0, (0, 2): 0,
                  (1, 0): 0, (1, 1): 0, (1, 2): 0}
        order = [(0, 1), (0, 0), (0, 2), (1, 0), (1, 1), (1, 2)]
        for _ in range(3):
            for key in order:
                g, c = key
                if hop_at[key] > PLANE - 2:
                    continue
                for i in range(2):
                    ag[(i, g, c)].wait()
                s = hop_at[key] + 1
                hop_at[key] = s
                if s <= PLANE - 2:
                    for i in range(2):
                        ag[(i, g, c)] = ag_hop(i, g, c, s)

        @functools.partial(pl.run_scoped, exit_sem=pltpu.SemaphoreType.REGULAR)
        def _(exit_sem):
            for nbr in neighbors:
                pl.semaphore_signal(exit_sem, inc=1, device_id=(nbr,),
                                    device_id_type=pl.DeviceIdType.MESH)
            pl.semaphore_wait(exit_sem, len(neighbors))

    return pl.pallas_call(
        body,
        out_shape=jax.ShapeDtypeStruct((m, n), jnp.bfloat16),
        in_specs=[
            pl.BlockSpec(memory_space=pltpu.VMEM),
            pl.BlockSpec(memory_space=pltpu.VMEM),
        ],
        out_specs=pl.BlockSpec(memory_space=pltpu.VMEM),
        scratch_shapes=[
            pltpu.VMEM((m, hn), jnp.bfloat16),
            pltpu.VMEM((m, hn), jnp.bfloat16),
            pltpu.VMEM((PLANE - 1, qr_rows, hg), jnp.bfloat16),
            pltpu.VMEM((PLANE - 1, qr_rows, hg), jnp.bfloat16),
            pltpu.VMEM((PLANE - 1, qr_rows, hg), jnp.bfloat16),
            pltpu.VMEM((PLANE - 1, qr_rows, hg), jnp.bfloat16),
            pltpu.VMEM((4, h2, hg), jnp.bfloat16),
            pltpu.VMEM((4, h4, hg), jnp.bfloat16),
            pltpu.SemaphoreType.DMA((6,)),
            pltpu.SemaphoreType.DMA((6,)),
            pltpu.SemaphoreType.DMA((6,)),
            pltpu.SemaphoreType.DMA((6,)),
            pltpu.SemaphoreType.DMA((20,)),
            pltpu.SemaphoreType.DMA((20,)),
            pltpu.SemaphoreType.DMA((18,)),
            pltpu.SemaphoreType.DMA((18,)),
            pltpu.SemaphoreType.DMA((18,)),
            pltpu.SemaphoreType.DMA((18,)),
        ],
        compiler_params=pltpu.CompilerParams(collective_id=0),
    )(A, B)


# device time: 71762 ns/iter; 1.0499x vs baseline; 1.0499x over previous
import functools

import jax
import jax.numpy as jnp
from jax import lax
from jax.experimental import pallas as pl
from jax.experimental.pallas import tpu as pltpu

N_DEV = 16
PLANE = 4
NZ = 4


def kernel(A, B):
    m, k = A.shape
    _, n = B.shape
    hn = n // 2
    qr_rows = m // PLANE
    h2 = qr_rows // 2
    h4 = qr_rows // 4

    def body(a_ref, b_ref, out_ref, acc_r, acc_l,
             px_buf_r, px_buf_l, py_buf_r, py_buf_l, sa_recv, sb_recv,
             p1_send_r, p1_recv_r, p1_send_l, p1_recv_l,
             p2_send, p2_recv,
             p3_send_r, p3_recv_r, p3_send_l, p3_recv_l):
        my = lax.axis_index("i")
        z = lax.div(my, PLANE)
        q = lax.rem(my, PLANE)
        qr = z * PLANE + lax.rem(q + 1, PLANE)
        ql = z * PLANE + lax.rem(q + PLANE - 1, PLANE)
        pz1 = q + PLANE * jnp.bitwise_xor(z, 1)
        pz2 = q + PLANE * jnp.bitwise_xor(z, 2)
        neighbors = (qr, ql, pz1, pz2)

        barrier = pltpu.get_barrier_semaphore()
        for nbr in neighbors:
            pl.semaphore_signal(barrier, inc=1, device_id=(nbr,),
                                device_id_type=pl.DeviceIdType.MESH)
        pl.semaphore_wait(barrier, len(neighbors))

        b_halves = [None, None]

        def compute_quarter(i, idx):
            a_q = a_ref[pl.ds(idx * qr_rows, qr_rows), :].astype(jnp.bfloat16)
            accs[i][pl.ds(idx * qr_rows, qr_rows), :] = jnp.dot(
                a_q, b_halves[i], preferred_element_type=jnp.float32
            ).astype(jnp.bfloat16)

        def rdma(src, dst, ssem, rsem, tgt):
            cp = pltpu.make_async_remote_copy(
                src_ref=src, dst_ref=dst, send_sem=ssem, recv_sem=rsem,
                device_id=(tgt,), device_id_type=pl.DeviceIdType.MESH,
            )
            cp.start()
            return cp

        b1 = jnp.bitwise_and(z, 1)
        b2 = jnp.bitwise_and(lax.div(z, 2), 1)
        keep1 = b1 * h2
        send1 = (1 - b1) * h2
        d_keep = b2 * h4
        d_send = (1 - b2) * h4
        keep2 = keep1 + d_keep
        send2 = keep1 + d_send

        accs = (acc_r, acc_l)
        xbufs = (px_buf_r, px_buf_l)
        ybufs = (py_buf_r, py_buf_l)
        p1s = (p1_send_r, p1_send_l)
        p1r = (p1_recv_r, p1_recv_l)
        tgt_ = (qr, ql)

        def chunk_q(i, s, delta):
            d = -s + delta if i == 0 else s + delta
            return lax.rem(q + d + 2 * PLANE, PLANE)

        def p1_hop(i, sub, s):
            off = send1 if sub == 0 else keep1
            buf = xbufs[i] if sub == 0 else ybufs[i]
            sem = 3 * sub + s
            src_q = chunk_q(i, s, 0)
            return rdma(
                accs[i].at[pl.ds(src_q * qr_rows + off, h2)], buf.at[s],
                p1s[i].at[sem], p1r[i].at[sem], tgt_[i])

        def p1_add(i, sub, s):
            off = send1 if sub == 0 else keep1
            buf = xbufs[i] if sub == 0 else ybufs[i]
            dst_q = chunk_q(i, s, -1 if i == 0 else 1)
            rows = pl.ds(dst_q * qr_rows + off, h2)
            accs[i][rows, :] = accs[i][rows, :] + buf[s]

        hops = {}
        for i, cols in enumerate((slice(0, hn), slice(hn, n))):
            b_halves[i] = b_ref[:, cols].astype(jnp.bfloat16)
            compute_quarter(i, q)
            hops[(i, 0)] = p1_hop(i, 0, 0)
            hops[(i, 1)] = p1_hop(i, 1, 0)

        base_r = lax.rem(q + 1, PLANE) * qr_rows
        base_l = lax.rem(q + PLANE - 1, PLANE) * qr_rows
        bases = (base_r, base_l)

        def z_reduce(i, step_idx, off, nrows, buf, tgt):
            return rdma(accs[i].at[pl.ds(bases[i] + off, nrows)],
                        buf,
                        p2_send.at[2 * step_idx + i],
                        p2_recv.at[2 * step_idx + i], tgt)

        def z_gather(i, step_idx, off, nrows, tgt):
            return rdma(accs[i].at[pl.ds(bases[i] + off, nrows)],
                        accs[i].at[pl.ds(bases[i] + off, nrows)],
                        p2_send.at[2 * step_idx + i],
                        p2_recv.at[2 * step_idx + i], tgt)

        za = {}
        for s in range(PLANE - 1):
            compute_quarter(0, chunk_q(0, s, -1))
            compute_quarter(1, chunk_q(1, s, 1))
            for i in range(2):
                hops[(i, 0)].wait()
                p1_add(i, 0, s)
                if s < PLANE - 2:
                    hops[(i, 0)] = p1_hop(i, 0, s + 1)
                else:
                    za[(i, 0)] = z_reduce(i, 0, send1 + d_send, h4,
                                          sa_recv.at[2 * i + 0], pz1)
                    za[(i, 1)] = z_reduce(i, 1, send1 + d_keep, h4,
                                          sa_recv.at[2 * i + 1], pz1)
            for i in range(2):
                hops[(i, 1)].wait()
                p1_add(i, 1, s)
                if s < PLANE - 2:
                    hops[(i, 1)] = p1_hop(i, 1, s + 1)

        zb = {}
        for i in range(2):
            za[(i, 0)].wait()
            rows = pl.ds(bases[i] + keep1 + d_send, h4)
            accs[i][rows, :] = accs[i][rows, :] + sa_recv[2 * i + 0]
        for i in range(2):
            zb[i] = z_reduce(i, 2, send2, h4, sb_recv.at[i], pz2)
            za[(i, 1)].wait()
            rows = pl.ds(bases[i] + keep1 + d_keep, h4)
            accs[i][rows, :] = accs[i][rows, :] + sa_recv[2 * i + 1]

        col_ = (pl.ds(0, hn), pl.ds(hn, hn))
        p3s = (p3_send_r, p3_send_l)
        p3r = (p3_recv_r, p3_recv_l)

        def ag_hop(i, chain, off, s):
            if s == 0:
                rows = pl.ds(bases[i] + off, h4)
                src = accs[i].at[rows]
            else:
                back = chunk_q(i, s - 1, 0)
                rows = pl.ds(back * qr_rows + off, h4)
                src = out_ref.at[rows, col_[i]]
            return rdma(src, out_ref.at[rows, col_[i]],
                        p3s[i].at[3 * chain + s],
                        p3r[i].at[3 * chain + s], tgt_[i])

        def out_own(i, off):
            rows = pl.ds(bases[i] + off, h4)
            out_ref[rows, col_[i]] = accs[i][rows, :]

        zc, zd1, zd2 = {}, {}, {}
        ag = {}
        for i in range(2):
            zb[i].wait()
            rows = pl.ds(bases[i] + keep2, h4)
            accs[i][rows, :] = jnp.maximum(
                accs[i][rows, :] + sb_recv[i], 0.0)
            zc[i] = z_gather(i, 3, keep2, h4, pz2)
            zd1[i] = z_gather(i, 4, keep2, h4, pz1)
            ag[(i, 0)] = ag_hop(i, 0, keep2, 0)
            out_own(i, keep2)

        for i in range(2):
            zc[i].wait()
            zd2[i] = z_gather(i, 5, send2, h4, pz1)
            ag[(i, 1)] = ag_hop(i, 1, send2, 0)
            out_own(i, send2)
        for i in range(2):
            ag[(i, 0)].wait()
            ag[(i, 0)] = ag_hop(i, 0, keep2, 1)
        for i in range(2):
            zd1[i].wait()
            ag[(i, 2)] = ag_hop(i, 2, send1 + d_keep, 0)
            out_own(i, send1 + d_keep)
        for i in range(2):
            zd2[i].wait()
            ag[(i, 3)] = ag_hop(i, 3, send1 + d_send, 0)
            out_own(i, send1 + d_send)

        offs = (keep2, send2, send1 + d_keep, send1 + d_send)
        hop_at = {0: 1, 1: 0, 2: 0, 3: 0}
        for chain in (1, 0, 2, 1, 3, 0, 2, 1, 3, 2, 3):
            for i in range(2):
                ag[(i, chain)].wait()
            s = hop_at[chain] + 1
            hop_at[chain] = s
            if s <= PLANE - 2:
                for i in range(2):
                    ag[(i, chain)] = ag_hop(i, chain, offs[chain], s)

        @functools.partial(pl.run_scoped, exit_sem=pltpu.SemaphoreType.REGULAR)
        def _(exit_sem):
            for nbr in neighbors:
                pl.semaphore_signal(exit_sem, inc=1, device_id=(nbr,),
                                    device_id_type=pl.DeviceIdType.MESH)
            pl.semaphore_wait(exit_sem, len(neighbors))

    return pl.pallas_call(
        body,
        out_shape=jax.ShapeDtypeStruct((m, n), jnp.bfloat16),
        in_specs=[
            pl.BlockSpec(memory_space=pltpu.VMEM),
            pl.BlockSpec(memory_space=pltpu.VMEM),
        ],
        out_specs=pl.BlockSpec(memory_space=pltpu.VMEM),
        scratch_shapes=[
            pltpu.VMEM((m, hn), jnp.bfloat16),
            pltpu.VMEM((m, hn), jnp.bfloat16),
            pltpu.VMEM((PLANE - 1, h2, hn), jnp.bfloat16),
            pltpu.VMEM((PLANE - 1, h2, hn), jnp.bfloat16),
            pltpu.VMEM((PLANE - 1, h2, hn), jnp.bfloat16),
            pltpu.VMEM((PLANE - 1, h2, hn), jnp.bfloat16),
            pltpu.VMEM((4, h4, hn), jnp.bfloat16),
            pltpu.VMEM((2, h4, hn), jnp.bfloat16),
            pltpu.SemaphoreType.DMA((6,)),
            pltpu.SemaphoreType.DMA((6,)),
            pltpu.SemaphoreType.DMA((6,)),
            pltpu.SemaphoreType.DMA((6,)),
            pltpu.SemaphoreType.DMA((12,)),
            pltpu.SemaphoreType.DMA((12,)),
            pltpu.SemaphoreType.DMA((12,)),
            pltpu.SemaphoreType.DMA((12,)),
            pltpu.SemaphoreType.DMA((12,)),
            pltpu.SemaphoreType.DMA((12,)),
        ],
        compiler_params=pltpu.CompilerParams(collective_id=0),
    )(A, B)
